# Initial kernel scaffold; baseline (speedup 1.0000x reference)
import jax
import jax.numpy as jnp
from jax.experimental import pallas as pl


def kernel(x, edge_index, batch, W_conv, b_conv, W_lin, b_lin):
    raise NotImplementedError("write your pallas kernel here")


# speedup vs baseline: 28.3314x; 28.3314x over previous
import jax
import jax.numpy as jnp
from jax import lax
from jax.experimental import pallas as pl
from jax.experimental.pallas import tpu as pltpu
from jax.experimental.pallas import tpu_sc as plsc

N = 10000
E = 320000
DIN = 128
DH = 128
DOUT = 64
G = 64

NC = 2
NS = 16
NW = NC * NS
EC = E // NW
CHUNK = 128
NFULL = EC // CHUNK
TAIL = EC - NFULL * CHUNK
NPAD = 10240
RPT = NPAD // NS
BLK = 2048

_mesh = plsc.VectorSubcoreMesh(core_axis_name="c", subcore_axis_name="s")


def _deg_body(dst_hbm, out_hbm, dst_v, deg_v):
    c = lax.axis_index("c")
    s = lax.axis_index("s")
    wid = s * NC + c
    base = wid * EC
    pltpu.sync_copy(dst_hbm.at[pl.ds(base, EC)], dst_v)
    z16 = jnp.zeros((16,), jnp.float32)
    ones16 = jnp.ones((16,), jnp.float32)

    def zero_body(i, carry):
        deg_v[pl.ds(i * 16, 16)] = z16
        return carry

    lax.fori_loop(0, NPAD // 16, zero_body, 0)

    def acc_body(i, carry):
        idx = dst_v[pl.ds(i * 16, 16)]
        plsc.addupdate_scatter(deg_v, [idx], ones16)
        return carry

    lax.fori_loop(0, EC // 16, acc_body, 0)
    pltpu.sync_copy(deg_v, out_hbm.at[wid])


_deg_call = pl.kernel(
    _deg_body,
    out_type=jax.ShapeDtypeStruct((NW, NPAD), jnp.float32),
    mesh=_mesh,
    compiler_params=pltpu.CompilerParams(needs_layout_passes=False),
    scratch_types=[
        pltpu.VMEM((EC,), jnp.int32),
        pltpu.VMEM((NPAD,), jnp.float32),
    ],
)


def _edge_body(src_hbm, dst_hbm, y_hbm, out_hbm, z_sh, sidx, didx, rows,
               sidx_t, didx_t, rows_t, gsem):
    c = lax.axis_index("c")
    s = lax.axis_index("s")
    wid = s * NC + c
    base = wid * EC

    z16 = jnp.zeros((16,), jnp.float32)

    def zrow(j, carry):
        def zcol(k, c2):
            rows[j, pl.ds(k * 16, 16)] = z16
            return c2

        lax.fori_loop(0, DH // 16, zcol, 0)
        return carry

    lax.fori_loop(0, CHUNK, zrow, 0)
    row0 = s * RPT
    for r in range(RPT // CHUNK):
        pltpu.sync_copy(rows, z_sh.at[pl.ds(row0 + r * CHUNK, CHUNK)])
    plsc.subcore_barrier()

    def chunk_body(i, carry):
        off = base + i * CHUNK
        pltpu.sync_copy(src_hbm.at[pl.ds(off, CHUNK)], sidx)
        cp = pltpu.async_copy(y_hbm.at[sidx], rows, gsem)
        pltpu.sync_copy(dst_hbm.at[pl.ds(off, CHUNK)], didx)
        cp.wait()
        pltpu.sync_copy(rows, z_sh.at[didx], add=True)
        return carry

    lax.fori_loop(0, NFULL, chunk_body, 0)

    off = base + NFULL * CHUNK
    pltpu.sync_copy(src_hbm.at[pl.ds(off, TAIL)], sidx_t)
    cp = pltpu.async_copy(y_hbm.at[sidx_t], rows_t, gsem)
    pltpu.sync_copy(dst_hbm.at[pl.ds(off, TAIL)], didx_t)
    cp.wait()
    pltpu.sync_copy(rows_t, z_sh.at[didx_t], add=True)

    plsc.subcore_barrier()
    pltpu.sync_copy(z_sh.at[pl.ds(row0, RPT)], out_hbm.at[c, pl.ds(row0, RPT)])


_edge_call = pl.kernel(
    _edge_body,
    out_type=jax.ShapeDtypeStruct((NC, NPAD, DH), jnp.float32),
    mesh=_mesh,
    scratch_types=[
        pltpu.VMEM_SHARED((NPAD, DH), jnp.float32),
        pltpu.VMEM((CHUNK,), jnp.int32),
        pltpu.VMEM((CHUNK,), jnp.int32),
        pltpu.VMEM((CHUNK, DH), jnp.float32),
        pltpu.VMEM((TAIL,), jnp.int32),
        pltpu.VMEM((TAIL,), jnp.int32),
        pltpu.VMEM((TAIL, DH), jnp.float32),
        pltpu.SemaphoreType.DMA,
    ],
)


def _tca_body(x_ref, w_ref, degt_ref, y_ref):
    deg = jnp.sum(degt_ref[...], axis=1, keepdims=True) + 1.0
    dinv = lax.rsqrt(deg)
    xw = jnp.dot(x_ref[...], w_ref[...], preferred_element_type=jnp.float32)
    y_ref[...] = xw * dinv


def _tca(x_p, w, degt):
    return pl.pallas_call(
        _tca_body,
        grid=(NPAD // BLK,),
        in_specs=[
            pl.BlockSpec((BLK, DIN), lambda i: (i, 0)),
            pl.BlockSpec((DIN, DH), lambda i: (0, 0)),
            pl.BlockSpec((BLK, NW), lambda i: (i, 0)),
        ],
        out_specs=pl.BlockSpec((BLK, DH), lambda i: (i, 0)),
        out_shape=jax.ShapeDtypeStruct((NPAD, DH), jnp.float32),
    )(x_p, w, degt)


def _tcb_body(z0_ref, z1_ref, y_ref, degt_ref, bconv_ref, batch_ref,
              wlin_ref, blin_ref, out_ref, sums, cnt):
    i = pl.program_id(0)

    @pl.when(i == 0)
    def _():
        sums[...] = jnp.zeros_like(sums)
        cnt[...] = jnp.zeros_like(cnt)

    deg = jnp.sum(degt_ref[...], axis=1, keepdims=True) + 1.0
    dinv = lax.rsqrt(deg)
    h = dinv * (z0_ref[...] + z1_ref[...] + y_ref[...]) + bconv_ref[...]
    h = jnp.maximum(h, 0.0)
    gid = lax.broadcasted_iota(jnp.int32, (BLK, G), 1)
    onehot = (batch_ref[...] == gid).astype(jnp.float32)
    sums[...] += lax.dot_general(onehot, h, (((0,), (0,)), ((), ())),
                                 preferred_element_type=jnp.float32)
    cnt[...] += lax.dot_general(onehot, jnp.ones((BLK, 1), jnp.float32),
                                (((0,), (0,)), ((), ())),
                                preferred_element_type=jnp.float32)

    @pl.when(i == pl.num_programs(0) - 1)
    def _():
        emb = sums[...] / jnp.maximum(cnt[...], 1.0)
        out_ref[...] = jnp.tanh(
            jnp.dot(emb, wlin_ref[...], preferred_element_type=jnp.float32)
            + blin_ref[...])


def _tcb(z0, z1, y, degt, bconv, batch2, wlin, blin):
    return pl.pallas_call(
        _tcb_body,
        grid=(NPAD // BLK,),
        in_specs=[
            pl.BlockSpec((BLK, DH), lambda i: (i, 0)),
            pl.BlockSpec((BLK, DH), lambda i: (i, 0)),
            pl.BlockSpec((BLK, DH), lambda i: (i, 0)),
            pl.BlockSpec((BLK, NW), lambda i: (i, 0)),
            pl.BlockSpec((1, DH), lambda i: (0, 0)),
            pl.BlockSpec((BLK, 1), lambda i: (i, 0)),
            pl.BlockSpec((DH, DOUT), lambda i: (0, 0)),
            pl.BlockSpec((1, DOUT), lambda i: (0, 0)),
        ],
        out_specs=pl.BlockSpec((G, DOUT), lambda i: (0, 0)),
        out_shape=jax.ShapeDtypeStruct((G, DOUT), jnp.float32),
        scratch_shapes=[
            pltpu.VMEM((G, DH), jnp.float32),
            pltpu.VMEM((G, 1), jnp.float32),
        ],
    )(z0, z1, y, degt, bconv, batch2, wlin, blin)


@jax.jit
def kernel(x, edge_index, batch, W_conv, b_conv, W_lin, b_lin):
    src = edge_index[0].astype(jnp.int32)
    dst = edge_index[1].astype(jnp.int32)

    degp = _deg_call(dst)
    degt = degp.T

    x_p = jnp.pad(x, ((0, NPAD - N), (0, 0)))
    y = _tca(x_p, W_conv, degt)

    zp = _edge_call(src, dst, y)

    batch2 = jnp.pad(batch.astype(jnp.int32), (0, NPAD - N),
                     constant_values=-1).reshape(NPAD, 1)
    bconv = b_conv.reshape(1, DH)
    blin = b_lin.reshape(1, DOUT)
    return _tcb(zp[0], zp[1], y, degt, bconv, batch2, W_lin, blin)
